# baseline (speedup 1.0000x reference)
import functools

import jax
import jax.numpy as jnp
from jax import lax
from jax.experimental import pallas as pl
from jax.experimental.pallas import tpu as pltpu
from jax.experimental.pallas import tpu_sc as plsc

BATCH = 16384
EMBED_DIM = 64
_NC = 2
_NS = 16
_NW = _NC * _NS
_BPW = BATCH // _NW
_CHUNK = 128
_NCHUNK = _BPW // _CHUNK


def _cf_body(uid_hbm, iid_hbm, ut_hbm, it_hbm, out_hbm,
             uidx_v, iidx_v, upair, ipair, ubuf, ibuf, out_v, sems):
    wid = lax.axis_index("s") * _NC + lax.axis_index("c")
    base = wid * _BPW

    pltpu.sync_copy(uid_hbm.at[pl.ds(base, _BPW)], uidx_v)
    pltpu.sync_copy(iid_hbm.at[pl.ds(base, _BPW)], iidx_v)

    def prep(j, b):
        for t in range(_CHUNK // 16):
            s = pl.ds(j * _CHUNK + t * 16, 16)
            d = pl.ds(t * 16, 16)
            upair[b].at[d][...] = uidx_v[s] >> 1
            ipair[b].at[d][...] = iidx_v[s] >> 1
        return (
            pltpu.async_copy(ut_hbm.at[upair[b]], ubuf[b], sems[2 * b]),
            pltpu.async_copy(it_hbm.at[ipair[b]], ibuf[b], sems[2 * b + 1]),
        )

    lanes = lax.iota(jnp.int32, 16)

    def compute(j, b):
        def group(g, _):
            rows = g * 16 + lanes
            s = pl.ds(j * _CHUNK + g * 16, 16)
            ucol = (uidx_v[s] & 1) * EMBED_DIM
            icol = (iidx_v[s] & 1) * EMBED_DIM
            acc = jnp.zeros((16,), jnp.float32)
            for d in range(EMBED_DIM):
                u = plsc.load_gather(ubuf[b], [rows, ucol + d])
                v = plsc.load_gather(ibuf[b], [rows, icol + d])
                acc = acc + u * v
            out_v[s] = acc
            return 0

        lax.fori_loop(0, _CHUNK // 16, group, 0)

    inflight = prep(0, 0)
    for j in range(_NCHUNK):
        b = j % 2
        cur = inflight
        if j + 1 < _NCHUNK:
            inflight = prep(j + 1, 1 - b)
        for cp in cur:
            cp.wait()
        compute(j, b)

    pltpu.sync_copy(out_v, out_hbm.at[pl.ds(base, _BPW)])


@jax.jit
def _cf_kernel(user_id, item_id, user_table, item_table):
    mesh = plsc.VectorSubcoreMesh(core_axis_name="c", subcore_axis_name="s")
    f = pl.kernel(
        _cf_body,
        out_type=jax.ShapeDtypeStruct((BATCH,), jnp.float32),
        mesh=mesh,
        scratch_types=[
            pltpu.VMEM((_BPW,), jnp.int32),
            pltpu.VMEM((_BPW,), jnp.int32),
            [pltpu.VMEM((_CHUNK,), jnp.int32) for _ in range(2)],
            [pltpu.VMEM((_CHUNK,), jnp.int32) for _ in range(2)],
            [pltpu.VMEM((_CHUNK, 2 * EMBED_DIM), jnp.float32) for _ in range(2)],
            [pltpu.VMEM((_CHUNK, 2 * EMBED_DIM), jnp.float32) for _ in range(2)],
            pltpu.VMEM((_BPW,), jnp.float32),
            [pltpu.SemaphoreType.DMA for _ in range(4)],
        ],
        compiler_params=pltpu.CompilerParams(
            needs_layout_passes=False, use_tc_tiling_on_sc=True),
    )
    ut2 = user_table.reshape(user_table.shape[0] // 2, 2 * EMBED_DIM)
    it2 = item_table.reshape(item_table.shape[0] // 2, 2 * EMBED_DIM)
    return f(user_id, item_id, ut2, it2)


def kernel(user_id, item_id, user_table, item_table):
    out = _cf_kernel(user_id, item_id, user_table, item_table)
    return out.reshape(BATCH, 1)
